# baseline (speedup 1.0000x reference)
import functools

import jax
import jax.numpy as jnp
from jax import lax
from jax.experimental import pallas as pl
from jax.experimental.pallas import tpu as pltpu
from jax.experimental.pallas import tpu_sc as plsc

H = 2048
L = 16
HL = H // L
NC = 2
NS = 16
NW = NC * NS
B = 4
S = 2048
TOKENS = B * S
TPW = TOKENS // NW
G = 16
NCH = TPW // G
WPB = NW // B
RH = 1024
NUM_SLOTS = 64
TOP_K = 8
MEM = 16

_mesh = plsc.VectorSubcoreMesh(core_axis_name="c", subcore_axis_name="s")


NB1 = 3


@functools.partial(
    pl.kernel,
    mesh=_mesh,
    out_type=jax.ShapeDtypeStruct((NW, H), jnp.float32),
    scratch_types=[
        pltpu.VMEM((TPW,), jnp.int32),
        pltpu.VMEM((G, H), jnp.float32),
        pltpu.VMEM((G, H), jnp.float32),
        pltpu.VMEM((G, H), jnp.float32),
        pltpu.VMEM((H,), jnp.float32),
        pltpu.SemaphoreType.DMA,
        pltpu.SemaphoreType.DMA,
        pltpu.SemaphoreType.DMA,
    ],
)
def _gather_partial_sums(ids_hbm, table_hbm, out_hbm, idx_all, rows0, rows1,
                         rows2, acc_v, gsem0, gsem1, gsem2):
    wid = lax.axis_index("s") * NC + lax.axis_index("c")
    base = wid * TPW
    bufs = [rows0, rows1, rows2]
    sems = [gsem0, gsem1, gsem2]

    pltpu.sync_copy(ids_hbm.at[pl.ds(base, TPW)], idx_all)

    def zero_j(j, c):
        acc_v[pl.ds(j * L, L)] = jnp.zeros((L,), jnp.float32)
        return c

    lax.fori_loop(0, HL, zero_j, 0)

    def g_start(ch):
        p = ch % NB1
        return pltpu.async_copy(
            table_hbm.at[idx_all.at[pl.ds(ch * G, G)]], bufs[p], sems[p])

    cps = {ch: g_start(ch) for ch in range(NB1 - 1)}
    for ch in range(NCH):
        p = ch % NB1
        cps[ch].wait()

        def acc_j(j, c):
            for jj in range(2):
                sl = pl.ds((j * 2 + jj) * L, L)
                vs = [bufs[p][r, sl] for r in range(G)]
                while len(vs) > 1:
                    nxt = [vs[i] + vs[i + 1] for i in range(0, len(vs) - 1, 2)]
                    if len(vs) % 2:
                        nxt.append(vs[-1])
                    vs = nxt
                acc_v[sl] = acc_v[sl] + vs[0]
            return c

        lax.fori_loop(0, HL // 2, acc_j, 0)
        if ch + NB1 - 1 < NCH:
            cps[ch + NB1 - 1] = g_start(ch + NB1 - 1)

    pltpu.sync_copy(acc_v, out_hbm.at[wid])


@functools.partial(
    pl.kernel,
    mesh=_mesh,
    out_type=jax.ShapeDtypeStruct((TOKENS, H), jnp.float32),
    scratch_types=[
        pltpu.VMEM((TPW,), jnp.int32),
        pltpu.VMEM((G, H), jnp.float32),
        pltpu.VMEM((G, H), jnp.float32),
        pltpu.VMEM((G, H), jnp.float32),
        pltpu.VMEM((H,), jnp.float32),
        pltpu.SemaphoreType.DMA,
        pltpu.SemaphoreType.DMA,
        pltpu.SemaphoreType.DMA,
        pltpu.SemaphoreType.DMA,
        pltpu.SemaphoreType.DMA,
        pltpu.SemaphoreType.DMA,
    ],
)
def _gather_add_scatter(ids_hbm, table_hbm, contrib_hbm, out_hbm,
                        idx_all, rows0, rows1, rows2, ctr_v,
                        gsem0, gsem1, gsem2, ssem0, ssem1, ssem2):
    wid = lax.axis_index("s") * NC + lax.axis_index("c")
    base = wid * TPW
    b = wid // WPB
    bufs = [rows0, rows1, rows2]
    gsems = [gsem0, gsem1, gsem2]
    ssems = [ssem0, ssem1, ssem2]

    pltpu.sync_copy(ids_hbm.at[pl.ds(base, TPW)], idx_all)
    pltpu.sync_copy(contrib_hbm.at[b], ctr_v)

    def g_start(ch):
        p = ch % 3
        return pltpu.async_copy(
            table_hbm.at[idx_all.at[pl.ds(ch * G, G)]], bufs[p], gsems[p])

    gcps = {0: g_start(0), 1: g_start(1)}
    scps = {}
    for ch in range(NCH):
        p = ch % 3
        gcps[ch].wait()

        def add_j(j, c):
            sl = pl.ds(j * L, L)
            cv = ctr_v[sl]
            for r in range(G):
                bufs[p][r, sl] = bufs[p][r, sl] + cv
            return c

        lax.fori_loop(0, HL, add_j, 0)
        scps[ch] = pltpu.async_copy(
            bufs[p], out_hbm.at[pl.ds(base + ch * G, G)], ssems[p])
        if ch + 2 < NCH:
            if ch - 1 >= 0:
                scps[ch - 1].wait()
            gcps[ch + 2] = g_start(ch + 2)

    scps[NCH - 2].wait()
    scps[NCH - 1].wait()


def _router_body(p_ref, w1_ref, b1_ref, w2_ref, b2_ref, gate_ref, mem_ref,
                 out_ref):
    q = p_ref[:, 0, :]
    for i in range(1, WPB):
        q = q + p_ref[:, i, :]
    q = q * jnp.float32(1.0 / S)

    h = jnp.dot(q, w1_ref[:], preferred_element_type=jnp.float32) + b1_ref[:]
    h = jnp.maximum(h, 0.0)
    logits = jnp.dot(h, w2_ref[:], preferred_element_type=jnp.float32) + b2_ref[:]

    iota = lax.broadcasted_iota(jnp.int32, (B, NUM_SLOTS), 1)
    vals = logits
    neg = jnp.float32(-1e30)
    tvs, ohs = [], []
    for _ in range(TOP_K):
        m = jnp.max(vals, axis=1, keepdims=True)
        fidx = jnp.min(jnp.where(vals >= m, iota, NUM_SLOTS), axis=1,
                       keepdims=True)
        oh = iota == fidx
        tvs.append(m)
        ohs.append(oh)
        vals = jnp.where(oh, neg, vals)

    m0 = tvs[0]
    es = [jnp.exp(t - m0) for t in tvs]
    denom = es[0]
    for e in es[1:]:
        denom = denom + e
    wf = jnp.zeros((B, NUM_SLOTS), jnp.float32)
    for e, oh in zip(es, ohs):
        wf = wf + jnp.where(oh, e / denom, 0.0)

    gs = 1.0 / (1.0 + jnp.exp(-gate_ref[:]))
    sc = gs[:, 0:1] * mem_ref[:, 0, :]
    for mm in range(1, MEM):
        sc = sc + gs[:, mm:mm + 1] * mem_ref[:, mm, :]

    out_ref[:] = jnp.dot(wf, sc, preferred_element_type=jnp.float32)


_router = pl.pallas_call(
    _router_body,
    out_shape=jax.ShapeDtypeStruct((B, H), jnp.float32),
)


def kernel(input_ids, embed_table, W1, b1, W2, b2, gate_logits, memory):
    ids_flat = input_ids.reshape(-1).astype(jnp.int32)
    partials = _gather_partial_sums(ids_flat, embed_table)
    p4 = partials.reshape(B, WPB, H)
    contrib = _router(p4, W1, b1.reshape(1, RH), W2, b2.reshape(1, NUM_SLOTS),
                      gate_logits, memory)
    out = _gather_add_scatter(ids_flat, embed_table, contrib)
    return out.reshape(B, S, H)


# speedup vs baseline: 1.0137x; 1.0137x over previous
import functools

import jax
import jax.numpy as jnp
from jax import lax
from jax.experimental import pallas as pl
from jax.experimental.pallas import tpu as pltpu
from jax.experimental.pallas import tpu_sc as plsc

H = 2048
L = 16
HL = H // L
NC = 2
NS = 16
NW = NC * NS
B = 4
S = 2048
TOKENS = B * S
TPW = TOKENS // NW
G = 16
NCH = TPW // G
WPB = NW // B
RH = 1024
NUM_SLOTS = 64
TOP_K = 8
MEM = 16

_mesh = plsc.VectorSubcoreMesh(core_axis_name="c", subcore_axis_name="s")


NB1 = 3


@functools.partial(
    pl.kernel,
    mesh=_mesh,
    out_type=jax.ShapeDtypeStruct((NW, H), jnp.float32),
    scratch_types=[
        pltpu.VMEM((TPW,), jnp.int32),
        pltpu.VMEM((G, H), jnp.float32),
        pltpu.VMEM((G, H), jnp.float32),
        pltpu.VMEM((G, H), jnp.float32),
        pltpu.VMEM((H,), jnp.float32),
        pltpu.SemaphoreType.DMA,
        pltpu.SemaphoreType.DMA,
        pltpu.SemaphoreType.DMA,
    ],
)
def _gather_partial_sums(ids_hbm, table_hbm, out_hbm, idx_all, rows0, rows1,
                         rows2, acc_v, gsem0, gsem1, gsem2):
    wid = lax.axis_index("s") * NC + lax.axis_index("c")
    base = wid * TPW
    bufs = [rows0, rows1, rows2]
    sems = [gsem0, gsem1, gsem2]

    pltpu.sync_copy(ids_hbm.at[pl.ds(base, TPW)], idx_all)

    def zero_j(j, c):
        acc_v[pl.ds(j * L, L)] = jnp.zeros((L,), jnp.float32)
        return c

    lax.fori_loop(0, HL, zero_j, 0)

    def g_start(ch):
        p = ch % NB1
        return pltpu.async_copy(
            table_hbm.at[idx_all.at[pl.ds(ch * G, G)]], bufs[p], sems[p])

    cps = {ch: g_start(ch) for ch in range(NB1 - 1)}
    for ch in range(NCH):
        p = ch % NB1
        cps[ch].wait()

        def acc_j(j, c):
            sl = pl.ds(j * L, L)
            vs = [bufs[p][r, sl] for r in range(G)]
            while len(vs) > 1:
                nxt = [vs[i] + vs[i + 1] for i in range(0, len(vs) - 1, 2)]
                if len(vs) % 2:
                    nxt.append(vs[-1])
                vs = nxt
            acc_v[sl] = acc_v[sl] + vs[0]
            return c

        lax.fori_loop(0, HL, acc_j, 0)
        if ch + NB1 - 1 < NCH:
            cps[ch + NB1 - 1] = g_start(ch + NB1 - 1)

    pltpu.sync_copy(acc_v, out_hbm.at[wid])


@functools.partial(
    pl.kernel,
    mesh=_mesh,
    out_type=jax.ShapeDtypeStruct((TOKENS, H), jnp.float32),
    scratch_types=[
        pltpu.VMEM((TPW,), jnp.int32),
        pltpu.VMEM((G, H), jnp.float32),
        pltpu.VMEM((G, H), jnp.float32),
        pltpu.VMEM((G, H), jnp.float32),
        pltpu.VMEM((H,), jnp.float32),
        pltpu.SemaphoreType.DMA,
        pltpu.SemaphoreType.DMA,
        pltpu.SemaphoreType.DMA,
        pltpu.SemaphoreType.DMA,
        pltpu.SemaphoreType.DMA,
        pltpu.SemaphoreType.DMA,
    ],
)
def _gather_add_scatter(ids_hbm, table_hbm, contrib_hbm, out_hbm,
                        idx_all, rows0, rows1, rows2, ctr_v,
                        gsem0, gsem1, gsem2, ssem0, ssem1, ssem2):
    wid = lax.axis_index("s") * NC + lax.axis_index("c")
    base = wid * TPW
    b = wid // WPB
    bufs = [rows0, rows1, rows2]
    gsems = [gsem0, gsem1, gsem2]
    ssems = [ssem0, ssem1, ssem2]

    pltpu.sync_copy(ids_hbm.at[pl.ds(base, TPW)], idx_all)
    pltpu.sync_copy(contrib_hbm.at[b], ctr_v)

    def g_start(ch):
        p = ch % 3
        return pltpu.async_copy(
            table_hbm.at[idx_all.at[pl.ds(ch * G, G)]], bufs[p], gsems[p])

    gcps = {0: g_start(0), 1: g_start(1)}
    scps = {}
    for ch in range(NCH):
        p = ch % 3
        gcps[ch].wait()

        def add_j(j, c):
            sl = pl.ds(j * L, L)
            cv = ctr_v[sl]
            for r in range(G):
                bufs[p][r, sl] = bufs[p][r, sl] + cv
            return c

        lax.fori_loop(0, HL, add_j, 0)
        scps[ch] = pltpu.async_copy(
            bufs[p], out_hbm.at[pl.ds(base + ch * G, G)], ssems[p])
        if ch + 2 < NCH:
            if ch - 1 >= 0:
                scps[ch - 1].wait()
            gcps[ch + 2] = g_start(ch + 2)

    scps[NCH - 2].wait()
    scps[NCH - 1].wait()


def _router_body(p_ref, w1_ref, b1_ref, w2_ref, b2_ref, gate_ref, mem_ref,
                 out_ref):
    q = p_ref[:, 0, :]
    for i in range(1, WPB):
        q = q + p_ref[:, i, :]
    q = q * jnp.float32(1.0 / S)

    h = jnp.dot(q, w1_ref[:], preferred_element_type=jnp.float32) + b1_ref[:]
    h = jnp.maximum(h, 0.0)
    logits = jnp.dot(h, w2_ref[:], preferred_element_type=jnp.float32) + b2_ref[:]

    iota = lax.broadcasted_iota(jnp.int32, (B, NUM_SLOTS), 1)
    vals = logits
    neg = jnp.float32(-1e30)
    tvs, ohs = [], []
    for _ in range(TOP_K):
        m = jnp.max(vals, axis=1, keepdims=True)
        fidx = jnp.min(jnp.where(vals >= m, iota, NUM_SLOTS), axis=1,
                       keepdims=True)
        oh = iota == fidx
        tvs.append(m)
        ohs.append(oh)
        vals = jnp.where(oh, neg, vals)

    m0 = tvs[0]
    es = [jnp.exp(t - m0) for t in tvs]
    denom = es[0]
    for e in es[1:]:
        denom = denom + e
    wf = jnp.zeros((B, NUM_SLOTS), jnp.float32)
    for e, oh in zip(es, ohs):
        wf = wf + jnp.where(oh, e / denom, 0.0)

    gs = 1.0 / (1.0 + jnp.exp(-gate_ref[:]))
    sc = gs[:, 0:1] * mem_ref[:, 0, :]
    for mm in range(1, MEM):
        sc = sc + gs[:, mm:mm + 1] * mem_ref[:, mm, :]

    out_ref[:] = jnp.dot(wf, sc, preferred_element_type=jnp.float32)


_router = pl.pallas_call(
    _router_body,
    out_shape=jax.ShapeDtypeStruct((B, H), jnp.float32),
)


def kernel(input_ids, embed_table, W1, b1, W2, b2, gate_logits, memory):
    ids_flat = input_ids.reshape(-1).astype(jnp.int32)
    partials = _gather_partial_sums(ids_flat, embed_table)
    p4 = partials.reshape(B, WPB, H)
    contrib = _router(p4, W1, b1.reshape(1, RH), W2, b2.reshape(1, NUM_SLOTS),
                      gate_logits, memory)
    out = _gather_add_scatter(ids_flat, embed_table, contrib)
    return out.reshape(B, S, H)
